# Initial kernel scaffold; baseline (speedup 1.0000x reference)
import jax
import jax.numpy as jnp
from jax.experimental import pallas as pl


def kernel(x, edge_index, W1, b1, W2, b2):
    raise NotImplementedError("write your pallas kernel here")


# speedup vs baseline: 35.4490x; 35.4490x over previous
import functools
import math

import jax
import jax.numpy as jnp
from jax import lax
from jax.experimental import pallas as pl
from jax.experimental.pallas import tpu as pltpu
from jax.experimental.pallas import tpu_sc as plsc

N = 10000
D = 128
E = 320000
HIDDEN = 64
C = 32
K = 10
ALPHA = 1.0
A_ = 1.0
B_ = 0.5
CC = 0.5

NTILES = 16
NCORES = 2
HALF = C // NCORES
N_PAD = 10240
CHUNK = N_PAD // NTILES
EBLK = 128
NBLK = 157
E_PAD = NTILES * NBLK * EBLK
ROWBLK = 512


def _gammas():
    cs = [1.0, (A_ - B_) / A_]
    for n in range(1, K):
        cs.append(((n + A_ - B_) * cs[n] - n * cs[n - 1]) / A_)
    return [math.exp(-ALPHA) * (ALPHA ** k) / math.factorial(k) * cs[k] * CC
            for k in range(K + 1)]


G = _gammas()


def _mlp_body(x_ref, w1_ref, b1_ref, w2_ref, b2_ref, o_ref):
    i = pl.program_id(0)
    h = jnp.dot(x_ref[...], w1_ref[...], preferred_element_type=jnp.float32)
    h = jnp.maximum(h + b1_ref[...], 0.0)
    h = jnp.dot(h, w2_ref[...], preferred_element_type=jnp.float32)
    h = h + b2_ref[...]
    rows = i * ROWBLK + lax.broadcasted_iota(jnp.int32, (ROWBLK, C), 0)
    o_ref[...] = jnp.where(rows < N, h, 0.0)


def _mlp(xp, W1, b1, W2, b2):
    grid = N_PAD // ROWBLK
    return pl.pallas_call(
        _mlp_body,
        grid=(grid,),
        in_specs=[
            pl.BlockSpec((ROWBLK, D), lambda i: (i, 0)),
            pl.BlockSpec((D, HIDDEN), lambda i: (0, 0)),
            pl.BlockSpec((1, HIDDEN), lambda i: (0, 0)),
            pl.BlockSpec((HIDDEN, C), lambda i: (0, 0)),
            pl.BlockSpec((1, C), lambda i: (0, 0)),
        ],
        out_specs=pl.BlockSpec((ROWBLK, C), lambda i: (i, 0)),
        out_shape=jax.ShapeDtypeStruct((N_PAD, C), jnp.float32),
    )(xp, W1, b1.reshape(1, HIDDEN), W2, b2.reshape(1, C))


def _lsm_body(x_ref, o_ref):
    x = x_ref[...]
    m = jnp.max(x, axis=1, keepdims=True)
    s = jnp.sum(jnp.exp(x - m), axis=1, keepdims=True)
    o_ref[...] = x - m - jnp.log(s)


def _lsm(o):
    grid = N_PAD // ROWBLK
    return pl.pallas_call(
        _lsm_body,
        grid=(grid,),
        in_specs=[pl.BlockSpec((ROWBLK, C), lambda i: (i, 0))],
        out_specs=pl.BlockSpec((ROWBLK, C), lambda i: (i, 0)),
        out_shape=jax.ShapeDtypeStruct((N_PAD, C), jnp.float32),
    )(o)


def _rsqrt16(d):
    i = lax.bitcast_convert_type(d, jnp.int32)
    i = 0x5F3759DF - lax.shift_right_arithmetic(i, 1)
    y = lax.bitcast_convert_type(i, jnp.float32)
    for _ in range(3):
        y = y * (1.5 - 0.5 * d * y * y)
    return y


def _prop_body(h_hbm, src_hbm, dst_hbm, out_hbm,
               t_sh, u_sh, deg_sh,
               src_v, dst_v, rowbuf, ones_v,
               nodebuf, outbuf, dqx, sdx, scal1, scal2):
    c = lax.axis_index("c")
    s = lax.axis_index("s")
    base = s * CHUNK

    pltpu.sync_copy(src_hbm.at[s], src_v)
    pltpu.sync_copy(dst_hbm.at[s], dst_v)

    def fill16(i, _):
        scal1[pl.ds(i * 16, 16)] = jnp.full((16,), 1.0, jnp.float32)
        return 0
    lax.fori_loop(0, CHUNK // 16, fill16, 0)

    def fillones(i, _):
        ones_v[pl.ds(i * 16, 16)] = jnp.full((16,), 1.0, jnp.float32)
        return 0
    lax.fori_loop(0, EBLK // 16, fillones, 0)

    pltpu.sync_copy(scal1, deg_sh.at[pl.ds(base, CHUNK)])
    plsc.subcore_barrier()

    def degblk(j, _):
        pltpu.sync_copy(ones_v, deg_sh.at[dst_v.at[j]], add=True)
        return 0
    lax.fori_loop(0, NBLK, degblk, 0)
    plsc.subcore_barrier()

    pltpu.sync_copy(deg_sh.at[pl.ds(base, CHUNK)], scal1)

    def newton(i, _):
        d = scal1[pl.ds(i * 16, 16)]
        y = _rsqrt16(d)
        scal1[pl.ds(i * 16, 16)] = y * y
        scal2[pl.ds(i * 16, 16)] = d * y
        return 0
    lax.fori_loop(0, CHUNK // 16, newton, 0)

    def expand(i, _):
        v1 = scal1[pl.ds(i * 16, 16)]
        v2 = scal2[pl.ds(i * 16, 16)]
        for l in range(16):
            dqx[i * 16 + l] = jnp.full((HALF,), v1[l], jnp.float32)
            sdx[i * 16 + l] = jnp.full((HALF,), v2[l], jnp.float32)
        return 0
    lax.fori_loop(0, CHUNK // 16, expand, 0)

    pltpu.sync_copy(h_hbm.at[c, pl.ds(base, CHUNK)], nodebuf)

    def init_row(r, _):
        hrow = nodebuf[r]
        dinv = dqx[r] * sdx[r]
        nodebuf[r] = hrow * dinv
        outbuf[r] = hrow * G[0]
        return 0
    lax.fori_loop(0, CHUNK, init_row, 0)

    pltpu.sync_copy(nodebuf, t_sh.at[pl.ds(base, CHUNK)])
    pltpu.sync_copy(nodebuf, u_sh.at[pl.ds(base, CHUNK)])
    plsc.subcore_barrier()

    for k in range(1, K + 1):
        def edge(j, _):
            pltpu.sync_copy(t_sh.at[src_v.at[j]], rowbuf)
            pltpu.sync_copy(rowbuf, u_sh.at[dst_v.at[j]], add=True)
            return 0
        lax.fori_loop(0, NBLK, edge, 0)
        plsc.subcore_barrier()

        pltpu.sync_copy(u_sh.at[pl.ds(base, CHUNK)], nodebuf)
        gk = G[k]

        def node(r, _):
            trow = nodebuf[r] * dqx[r]
            nodebuf[r] = trow
            outbuf[r] = outbuf[r] + trow * sdx[r] * gk
            return 0
        lax.fori_loop(0, CHUNK, node, 0)

        pltpu.sync_copy(nodebuf, t_sh.at[pl.ds(base, CHUNK)])
        if k < K:
            pltpu.sync_copy(nodebuf, u_sh.at[pl.ds(base, CHUNK)])
        plsc.subcore_barrier()

    pltpu.sync_copy(outbuf, out_hbm.at[c, pl.ds(base, CHUNK)])


def _prop(h2, srcb, dstb):
    mesh = plsc.VectorSubcoreMesh(
        core_axis_name="c", subcore_axis_name="s",
        num_cores=NCORES, num_subcores=NTILES)
    f = pl.kernel(
        _prop_body,
        out_type=jax.ShapeDtypeStruct((NCORES, N_PAD, HALF), jnp.float32),
        mesh=mesh,
        compiler_params=pltpu.CompilerParams(use_tc_tiling_on_sc=False),
        scratch_types=[
            pltpu.VMEM_SHARED((N_PAD, HALF), jnp.float32),
            pltpu.VMEM_SHARED((N_PAD, HALF), jnp.float32),
            pltpu.VMEM_SHARED((N_PAD,), jnp.float32),
            pltpu.VMEM((NBLK, EBLK), jnp.int32),
            pltpu.VMEM((NBLK, EBLK), jnp.int32),
            pltpu.VMEM((EBLK, HALF), jnp.float32),
            pltpu.VMEM((EBLK,), jnp.float32),
            pltpu.VMEM((CHUNK, HALF), jnp.float32),
            pltpu.VMEM((CHUNK, HALF), jnp.float32),
            pltpu.VMEM((CHUNK, HALF), jnp.float32),
            pltpu.VMEM((CHUNK, HALF), jnp.float32),
            pltpu.VMEM((CHUNK,), jnp.float32),
            pltpu.VMEM((CHUNK,), jnp.float32),
        ],
    )
    return f(h2, srcb, dstb)


def kernel(x, edge_index, W1, b1, W2, b2):
    xp = jnp.pad(x, ((0, N_PAD - N), (0, 0)))
    h = _mlp(xp, W1, b1, W2, b2)
    h2 = h.reshape(N_PAD, NCORES, HALF).transpose(1, 0, 2)

    pad_ids = N + (jnp.arange(E_PAD - E, dtype=jnp.int32) % (N_PAD - N))
    srcb = jnp.concatenate([edge_index[0], pad_ids]).reshape(NTILES, NBLK, EBLK)
    dstb = jnp.concatenate([edge_index[1], pad_ids]).reshape(NTILES, NBLK, EBLK)

    out2 = _prop(h2, srcb, dstb)
    out = out2.transpose(1, 0, 2).reshape(N_PAD, C)
    return _lsm(out)[:N]


# speedup vs baseline: 47.9306x; 1.3521x over previous
import functools
import math

import jax
import jax.numpy as jnp
from jax import lax
from jax.experimental import pallas as pl
from jax.experimental.pallas import tpu as pltpu
from jax.experimental.pallas import tpu_sc as plsc

N = 10000
D = 128
E = 320000
HIDDEN = 64
C = 32
K = 10
ALPHA = 1.0
A_ = 1.0
B_ = 0.5
CC = 0.5

NTILES = 16
NCORES = 2
HALF = C // NCORES
N_PAD = 10240
CHUNK = N_PAD // NTILES
EBLK = 128
NBLK = 157
E_PAD = NTILES * NBLK * EBLK
ROWBLK = 512


def _gammas():
    cs = [1.0, (A_ - B_) / A_]
    for n in range(1, K):
        cs.append(((n + A_ - B_) * cs[n] - n * cs[n - 1]) / A_)
    return [math.exp(-ALPHA) * (ALPHA ** k) / math.factorial(k) * cs[k] * CC
            for k in range(K + 1)]


G = _gammas()


def _mlp_body(x_ref, w1_ref, b1_ref, w2_ref, b2_ref, o_ref):
    i = pl.program_id(0)
    h = jnp.dot(x_ref[...], w1_ref[...], preferred_element_type=jnp.float32)
    h = jnp.maximum(h + b1_ref[...], 0.0)
    h = jnp.dot(h, w2_ref[...], preferred_element_type=jnp.float32)
    h = h + b2_ref[...]
    rows = i * ROWBLK + lax.broadcasted_iota(jnp.int32, (ROWBLK, C), 0)
    o_ref[...] = jnp.where(rows < N, h, 0.0)


def _mlp(xp, W1, b1, W2, b2):
    grid = N_PAD // ROWBLK
    return pl.pallas_call(
        _mlp_body,
        grid=(grid,),
        in_specs=[
            pl.BlockSpec((ROWBLK, D), lambda i: (i, 0)),
            pl.BlockSpec((D, HIDDEN), lambda i: (0, 0)),
            pl.BlockSpec((1, HIDDEN), lambda i: (0, 0)),
            pl.BlockSpec((HIDDEN, C), lambda i: (0, 0)),
            pl.BlockSpec((1, C), lambda i: (0, 0)),
        ],
        out_specs=pl.BlockSpec((ROWBLK, C), lambda i: (i, 0)),
        out_shape=jax.ShapeDtypeStruct((N_PAD, C), jnp.float32),
    )(xp, W1, b1.reshape(1, HIDDEN), W2, b2.reshape(1, C))


def _lsm_body(x_ref, o_ref):
    x = x_ref[...]
    m = jnp.max(x, axis=1, keepdims=True)
    s = jnp.sum(jnp.exp(x - m), axis=1, keepdims=True)
    o_ref[...] = x - m - jnp.log(s)


def _lsm(o):
    grid = N_PAD // ROWBLK
    return pl.pallas_call(
        _lsm_body,
        grid=(grid,),
        in_specs=[pl.BlockSpec((ROWBLK, C), lambda i: (i, 0))],
        out_specs=pl.BlockSpec((ROWBLK, C), lambda i: (i, 0)),
        out_shape=jax.ShapeDtypeStruct((N_PAD, C), jnp.float32),
    )(o)


def _rsqrt16(d):
    i = lax.bitcast_convert_type(d, jnp.int32)
    i = 0x5F3759DF - lax.shift_right_arithmetic(i, 1)
    y = lax.bitcast_convert_type(i, jnp.float32)
    for _ in range(3):
        y = y * (1.5 - 0.5 * d * y * y)
    return y


def _prop_body(h_hbm, src_hbm, dst_hbm, out_hbm,
               t_sh, u_sh, deg_sh,
               src_v, dst_v, rowbuf, ones_v,
               nodebuf, outbuf, dqx, sdx, scal1, scal2, gsem, ssem):
    c = lax.axis_index("c")
    s = lax.axis_index("s")
    base = s * CHUNK

    pltpu.sync_copy(src_hbm.at[s], src_v)
    pltpu.sync_copy(dst_hbm.at[s], dst_v)

    def fill16(i, _):
        scal1[pl.ds(i * 16, 16)] = jnp.full((16,), 1.0, jnp.float32)
        return 0
    lax.fori_loop(0, CHUNK // 16, fill16, 0)

    def fillones(i, _):
        ones_v[pl.ds(i * 16, 16)] = jnp.full((16,), 1.0, jnp.float32)
        return 0
    lax.fori_loop(0, EBLK // 16, fillones, 0)

    pltpu.sync_copy(scal1, deg_sh.at[pl.ds(base, CHUNK)])
    plsc.subcore_barrier()

    def degblk(j, _):
        pltpu.sync_copy(ones_v, deg_sh.at[dst_v.at[j]], add=True)
        return 0
    lax.fori_loop(0, NBLK, degblk, 0)
    plsc.subcore_barrier()

    pltpu.sync_copy(deg_sh.at[pl.ds(base, CHUNK)], scal1)

    def newton(i, _):
        d = scal1[pl.ds(i * 16, 16)]
        y = _rsqrt16(d)
        scal1[pl.ds(i * 16, 16)] = y * y
        scal2[pl.ds(i * 16, 16)] = d * y
        return 0
    lax.fori_loop(0, CHUNK // 16, newton, 0)

    def expand(i, _):
        v1 = scal1[pl.ds(i * 16, 16)]
        v2 = scal2[pl.ds(i * 16, 16)]
        for l in range(16):
            dqx[i * 16 + l] = jnp.full((HALF,), v1[l], jnp.float32)
            sdx[i * 16 + l] = jnp.full((HALF,), v2[l], jnp.float32)
        return 0
    lax.fori_loop(0, CHUNK // 16, expand, 0)

    pltpu.sync_copy(h_hbm.at[c, pl.ds(base, CHUNK)], nodebuf)

    def init_row(r, _):
        hrow = nodebuf[r]
        dinv = dqx[r] * sdx[r]
        nodebuf[r] = hrow * dinv
        outbuf[r] = hrow * G[0]
        return 0
    lax.fori_loop(0, CHUNK, init_row, 0)

    pltpu.sync_copy(nodebuf, t_sh.at[pl.ds(base, CHUNK)])
    pltpu.sync_copy(nodebuf, u_sh.at[pl.ds(base, CHUNK)])
    plsc.subcore_barrier()

    for k in range(1, K + 1):
        pltpu.async_copy(t_sh.at[src_v.at[0]], rowbuf.at[0], gsem)

        def edge(j, _):
            @pl.when(j >= 1)
            def _():
                pltpu.make_async_copy(
                    rowbuf.at[(j - 1) % 2],
                    u_sh.at[dst_v.at[j - 1]], ssem).wait()

            @pl.when(j + 1 < NBLK)
            def _():
                pltpu.async_copy(
                    t_sh.at[src_v.at[j + 1]], rowbuf.at[(j + 1) % 2], gsem)

            pltpu.make_async_copy(
                t_sh.at[src_v.at[j]], rowbuf.at[j % 2], gsem).wait()
            pltpu.async_copy(
                rowbuf.at[j % 2], u_sh.at[dst_v.at[j]], ssem, add=True)
            return 0
        lax.fori_loop(0, NBLK, edge, 0)
        pltpu.make_async_copy(
            rowbuf.at[(NBLK - 1) % 2],
            u_sh.at[dst_v.at[NBLK - 1]], ssem).wait()
        plsc.subcore_barrier()

        pltpu.sync_copy(u_sh.at[pl.ds(base, CHUNK)], nodebuf)
        gk = G[k]

        def node(r, _):
            trow = nodebuf[r] * dqx[r]
            nodebuf[r] = trow
            outbuf[r] = outbuf[r] + trow * sdx[r] * gk
            return 0
        lax.fori_loop(0, CHUNK, node, 0)

        pltpu.sync_copy(nodebuf, t_sh.at[pl.ds(base, CHUNK)])
        if k < K:
            pltpu.sync_copy(nodebuf, u_sh.at[pl.ds(base, CHUNK)])
        plsc.subcore_barrier()

    pltpu.sync_copy(outbuf, out_hbm.at[c, pl.ds(base, CHUNK)])


def _prop(h2, srcb, dstb):
    mesh = plsc.VectorSubcoreMesh(
        core_axis_name="c", subcore_axis_name="s",
        num_cores=NCORES, num_subcores=NTILES)
    f = pl.kernel(
        _prop_body,
        out_type=jax.ShapeDtypeStruct((NCORES, N_PAD, HALF), jnp.float32),
        mesh=mesh,
        compiler_params=pltpu.CompilerParams(use_tc_tiling_on_sc=False),
        scratch_types=[
            pltpu.VMEM_SHARED((N_PAD, HALF), jnp.float32),
            pltpu.VMEM_SHARED((N_PAD, HALF), jnp.float32),
            pltpu.VMEM_SHARED((N_PAD,), jnp.float32),
            pltpu.VMEM((NBLK, EBLK), jnp.int32),
            pltpu.VMEM((NBLK, EBLK), jnp.int32),
            pltpu.VMEM((2, EBLK, HALF), jnp.float32),
            pltpu.VMEM((EBLK,), jnp.float32),
            pltpu.VMEM((CHUNK, HALF), jnp.float32),
            pltpu.VMEM((CHUNK, HALF), jnp.float32),
            pltpu.VMEM((CHUNK, HALF), jnp.float32),
            pltpu.VMEM((CHUNK, HALF), jnp.float32),
            pltpu.VMEM((CHUNK,), jnp.float32),
            pltpu.VMEM((CHUNK,), jnp.float32),
            pltpu.SemaphoreType.DMA,
            pltpu.SemaphoreType.DMA,
        ],
    )
    return f(h2, srcb, dstb)


def kernel(x, edge_index, W1, b1, W2, b2):
    xp = jnp.pad(x, ((0, N_PAD - N), (0, 0)))
    h = _mlp(xp, W1, b1, W2, b2)
    h2 = h.reshape(N_PAD, NCORES, HALF).transpose(1, 0, 2)

    pad_ids = N + (jnp.arange(E_PAD - E, dtype=jnp.int32) % (N_PAD - N))
    srcb = jnp.concatenate([edge_index[0], pad_ids]).reshape(NTILES, NBLK, EBLK)
    dstb = jnp.concatenate([edge_index[1], pad_ids]).reshape(NTILES, NBLK, EBLK)

    out2 = _prop(h2, srcb, dstb)
    out = out2.transpose(1, 0, 2).reshape(N_PAD, C)
    return _lsm(out)[:N]


# speedup vs baseline: 57.7279x; 1.2044x over previous
import functools
import math

import jax
import jax.numpy as jnp
from jax import lax
from jax.experimental import pallas as pl
from jax.experimental.pallas import tpu as pltpu
from jax.experimental.pallas import tpu_sc as plsc

N = 10000
D = 128
E = 320000
HIDDEN = 64
C = 32
K = 10
ALPHA = 1.0
A_ = 1.0
B_ = 0.5
CC = 0.5

NTILES = 16
NCORES = 2
HALF = C // NCORES
N_PAD = 10240
CHUNK = N_PAD // NTILES
EBLK = 128
NBLK = 157
E_PAD = NTILES * NBLK * EBLK
ROWBLK = 512
NBUF = 4
AHEAD = NBUF // 2


def _gammas():
    cs = [1.0, (A_ - B_) / A_]
    for n in range(1, K):
        cs.append(((n + A_ - B_) * cs[n] - n * cs[n - 1]) / A_)
    return [math.exp(-ALPHA) * (ALPHA ** k) / math.factorial(k) * cs[k] * CC
            for k in range(K + 1)]


G = _gammas()


def _mlp_body(x_ref, w1_ref, b1_ref, w2_ref, b2_ref, o_ref):
    i = pl.program_id(0)
    h = jnp.dot(x_ref[...], w1_ref[...], preferred_element_type=jnp.float32)
    h = jnp.maximum(h + b1_ref[...], 0.0)
    h = jnp.dot(h, w2_ref[...], preferred_element_type=jnp.float32)
    h = h + b2_ref[...]
    rows = i * ROWBLK + lax.broadcasted_iota(jnp.int32, (ROWBLK, C), 0)
    o_ref[...] = jnp.where(rows < N, h, 0.0)


def _mlp(xp, W1, b1, W2, b2):
    grid = N_PAD // ROWBLK
    return pl.pallas_call(
        _mlp_body,
        grid=(grid,),
        in_specs=[
            pl.BlockSpec((ROWBLK, D), lambda i: (i, 0)),
            pl.BlockSpec((D, HIDDEN), lambda i: (0, 0)),
            pl.BlockSpec((1, HIDDEN), lambda i: (0, 0)),
            pl.BlockSpec((HIDDEN, C), lambda i: (0, 0)),
            pl.BlockSpec((1, C), lambda i: (0, 0)),
        ],
        out_specs=pl.BlockSpec((ROWBLK, C), lambda i: (i, 0)),
        out_shape=jax.ShapeDtypeStruct((N_PAD, C), jnp.float32),
    )(xp, W1, b1.reshape(1, HIDDEN), W2, b2.reshape(1, C))


def _lsm_body(x_ref, o_ref):
    x = x_ref[...]
    m = jnp.max(x, axis=1, keepdims=True)
    s = jnp.sum(jnp.exp(x - m), axis=1, keepdims=True)
    o_ref[...] = x - m - jnp.log(s)


def _lsm(o):
    grid = N_PAD // ROWBLK
    return pl.pallas_call(
        _lsm_body,
        grid=(grid,),
        in_specs=[pl.BlockSpec((ROWBLK, C), lambda i: (i, 0))],
        out_specs=pl.BlockSpec((ROWBLK, C), lambda i: (i, 0)),
        out_shape=jax.ShapeDtypeStruct((N_PAD, C), jnp.float32),
    )(o)


def _rsqrt16(d):
    i = lax.bitcast_convert_type(d, jnp.int32)
    i = 0x5F3759DF - lax.shift_right_arithmetic(i, 1)
    y = lax.bitcast_convert_type(i, jnp.float32)
    for _ in range(3):
        y = y * (1.5 - 0.5 * d * y * y)
    return y


def _prop_body(h_hbm, src_hbm, dst_hbm, out_hbm,
               t_sh, u_sh, deg_sh,
               src_v, dst_v, rowbuf, ones_v,
               nodebuf, outbuf, dqx, sdx, scal1, scal2, gsem, ssem):
    c = lax.axis_index("c")
    s = lax.axis_index("s")
    base = s * CHUNK

    pltpu.sync_copy(src_hbm.at[s], src_v)
    pltpu.sync_copy(dst_hbm.at[s], dst_v)

    def fill16(i, _):
        scal1[pl.ds(i * 16, 16)] = jnp.full((16,), 1.0, jnp.float32)
        return 0
    lax.fori_loop(0, CHUNK // 16, fill16, 0)

    def fillones(i, _):
        ones_v[pl.ds(i * 16, 16)] = jnp.full((16,), 1.0, jnp.float32)
        return 0
    lax.fori_loop(0, EBLK // 16, fillones, 0)

    pltpu.sync_copy(scal1, deg_sh.at[pl.ds(base, CHUNK)])
    plsc.subcore_barrier()

    def degblk(j, _):
        pltpu.sync_copy(ones_v, deg_sh.at[dst_v.at[j]], add=True)
        return 0
    lax.fori_loop(0, NBLK, degblk, 0)
    plsc.subcore_barrier()

    pltpu.sync_copy(deg_sh.at[pl.ds(base, CHUNK)], scal1)

    def newton(i, _):
        d = scal1[pl.ds(i * 16, 16)]
        y = _rsqrt16(d)
        scal1[pl.ds(i * 16, 16)] = y * y
        scal2[pl.ds(i * 16, 16)] = d * y
        return 0
    lax.fori_loop(0, CHUNK // 16, newton, 0)

    def expand(i, _):
        v1 = scal1[pl.ds(i * 16, 16)]
        v2 = scal2[pl.ds(i * 16, 16)]
        for l in range(16):
            dqx[i * 16 + l] = jnp.full((HALF,), v1[l], jnp.float32)
            sdx[i * 16 + l] = jnp.full((HALF,), v2[l], jnp.float32)
        return 0
    lax.fori_loop(0, CHUNK // 16, expand, 0)

    pltpu.sync_copy(h_hbm.at[c, pl.ds(base, CHUNK)], nodebuf)

    def init_row(r, _):
        hrow = nodebuf[r]
        dinv = dqx[r] * sdx[r]
        nodebuf[r] = hrow * dinv
        outbuf[r] = hrow * G[0]
        return 0
    lax.fori_loop(0, CHUNK, init_row, 0)

    pltpu.sync_copy(nodebuf, t_sh.at[pl.ds(base, CHUNK)])
    pltpu.sync_copy(nodebuf, u_sh.at[pl.ds(base, CHUNK)])
    plsc.subcore_barrier()

    for k in range(1, K + 1):
        for p in range(AHEAD):
            pltpu.async_copy(t_sh.at[src_v.at[p]], rowbuf.at[p], gsem)

        def edge(j, _):
            @pl.when(j >= AHEAD)
            def _():
                jj = j - AHEAD
                pltpu.make_async_copy(
                    rowbuf.at[jj % NBUF],
                    u_sh.at[dst_v.at[jj]], ssem).wait()

            @pl.when(j + AHEAD < NBLK)
            def _():
                jn = j + AHEAD
                pltpu.async_copy(
                    t_sh.at[src_v.at[jn]], rowbuf.at[jn % NBUF], gsem)

            pltpu.make_async_copy(
                t_sh.at[src_v.at[j]], rowbuf.at[j % NBUF], gsem).wait()
            pltpu.async_copy(
                rowbuf.at[j % NBUF], u_sh.at[dst_v.at[j]], ssem, add=True)
            return 0
        lax.fori_loop(0, NBLK, edge, 0)
        for p in range(AHEAD):
            jj = NBLK - AHEAD + p
            pltpu.make_async_copy(
                rowbuf.at[jj % NBUF], u_sh.at[dst_v.at[jj]], ssem).wait()
        plsc.subcore_barrier()

        pltpu.sync_copy(u_sh.at[pl.ds(base, CHUNK)], nodebuf)
        gk = G[k]

        def node(r, _):
            trow = nodebuf[r] * dqx[r]
            nodebuf[r] = trow
            outbuf[r] = outbuf[r] + trow * sdx[r] * gk
            return 0
        lax.fori_loop(0, CHUNK, node, 0)

        pltpu.sync_copy(nodebuf, t_sh.at[pl.ds(base, CHUNK)])
        if k < K:
            pltpu.sync_copy(nodebuf, u_sh.at[pl.ds(base, CHUNK)])
        plsc.subcore_barrier()

    pltpu.sync_copy(outbuf, out_hbm.at[c, pl.ds(base, CHUNK)])


def _prop(h2, srcb, dstb):
    mesh = plsc.VectorSubcoreMesh(
        core_axis_name="c", subcore_axis_name="s",
        num_cores=NCORES, num_subcores=NTILES)
    f = pl.kernel(
        _prop_body,
        out_type=jax.ShapeDtypeStruct((NCORES, N_PAD, HALF), jnp.float32),
        mesh=mesh,
        compiler_params=pltpu.CompilerParams(use_tc_tiling_on_sc=False),
        scratch_types=[
            pltpu.VMEM_SHARED((N_PAD, HALF), jnp.float32),
            pltpu.VMEM_SHARED((N_PAD, HALF), jnp.float32),
            pltpu.VMEM_SHARED((N_PAD,), jnp.float32),
            pltpu.VMEM((NBLK, EBLK), jnp.int32),
            pltpu.VMEM((NBLK, EBLK), jnp.int32),
            pltpu.VMEM((NBUF, EBLK, HALF), jnp.float32),
            pltpu.VMEM((EBLK,), jnp.float32),
            pltpu.VMEM((CHUNK, HALF), jnp.float32),
            pltpu.VMEM((CHUNK, HALF), jnp.float32),
            pltpu.VMEM((CHUNK, HALF), jnp.float32),
            pltpu.VMEM((CHUNK, HALF), jnp.float32),
            pltpu.VMEM((CHUNK,), jnp.float32),
            pltpu.VMEM((CHUNK,), jnp.float32),
            pltpu.SemaphoreType.DMA,
            pltpu.SemaphoreType.DMA,
        ],
    )
    return f(h2, srcb, dstb)


def kernel(x, edge_index, W1, b1, W2, b2):
    xp = jnp.pad(x, ((0, N_PAD - N), (0, 0)))
    h = _mlp(xp, W1, b1, W2, b2)
    h2 = h.reshape(N_PAD, NCORES, HALF).transpose(1, 0, 2)

    pad_ids = N + (jnp.arange(E_PAD - E, dtype=jnp.int32) % (N_PAD - N))
    srcb = jnp.concatenate([edge_index[0], pad_ids]).reshape(NTILES, NBLK, EBLK)
    dstb = jnp.concatenate([edge_index[1], pad_ids]).reshape(NTILES, NBLK, EBLK)

    out2 = _prop(h2, srcb, dstb)
    out = out2.transpose(1, 0, 2).reshape(N_PAD, C)
    return _lsm(out)[:N]


# speedup vs baseline: 60.4382x; 1.0469x over previous
import functools
import math

import jax
import jax.numpy as jnp
from jax import lax
from jax.experimental import pallas as pl
from jax.experimental.pallas import tpu as pltpu
from jax.experimental.pallas import tpu_sc as plsc

N = 10000
D = 128
E = 320000
HIDDEN = 64
C = 32
K = 10
ALPHA = 1.0
A_ = 1.0
B_ = 0.5
CC = 0.5

NTILES = 16
NCORES = 2
HALF = C // NCORES
N_PAD = 10240
CHUNK = N_PAD // NTILES
EBLK = 128
NBLK = 157
E_PAD = NTILES * NBLK * EBLK
ROWBLK = 512
NBUF = 4
AHEAD = NBUF // 2


def _gammas():
    cs = [1.0, (A_ - B_) / A_]
    for n in range(1, K):
        cs.append(((n + A_ - B_) * cs[n] - n * cs[n - 1]) / A_)
    return [math.exp(-ALPHA) * (ALPHA ** k) / math.factorial(k) * cs[k] * CC
            for k in range(K + 1)]


G = _gammas()


def _mlp_body(x_ref, w1_ref, b1_ref, w2_ref, b2_ref, o_ref):
    i = pl.program_id(0)
    h = jnp.dot(x_ref[...], w1_ref[...], preferred_element_type=jnp.float32)
    h = jnp.maximum(h + b1_ref[...], 0.0)
    h = jnp.dot(h, w2_ref[...], preferred_element_type=jnp.float32)
    h = h + b2_ref[...]
    rows = i * ROWBLK + lax.broadcasted_iota(jnp.int32, (ROWBLK, C), 0)
    h = jnp.where(rows < N, h, 0.0)
    o_ref[0] = h[:, :HALF]
    o_ref[1] = h[:, HALF:]


def _mlp(x, W1, b1, W2, b2):
    grid = N_PAD // ROWBLK
    return pl.pallas_call(
        _mlp_body,
        grid=(grid,),
        in_specs=[
            pl.BlockSpec((ROWBLK, D), lambda i: (i, 0)),
            pl.BlockSpec((D, HIDDEN), lambda i: (0, 0)),
            pl.BlockSpec((1, HIDDEN), lambda i: (0, 0)),
            pl.BlockSpec((HIDDEN, C), lambda i: (0, 0)),
            pl.BlockSpec((1, C), lambda i: (0, 0)),
        ],
        out_specs=pl.BlockSpec((NCORES, ROWBLK, HALF), lambda i: (0, i, 0)),
        out_shape=jax.ShapeDtypeStruct((NCORES, N_PAD, HALF), jnp.float32),
    )(x, W1, b1.reshape(1, HIDDEN), W2, b2.reshape(1, C))


def _lsm_body(x_ref, o_ref):
    x = jnp.concatenate([x_ref[0], x_ref[1]], axis=1)
    m = jnp.max(x, axis=1, keepdims=True)
    s = jnp.sum(jnp.exp(x - m), axis=1, keepdims=True)
    o_ref[...] = x - m - jnp.log(s)


def _lsm(o2):
    grid = N_PAD // ROWBLK
    return pl.pallas_call(
        _lsm_body,
        grid=(grid,),
        in_specs=[pl.BlockSpec((NCORES, ROWBLK, HALF), lambda i: (0, i, 0))],
        out_specs=pl.BlockSpec((ROWBLK, C), lambda i: (i, 0)),
        out_shape=jax.ShapeDtypeStruct((N, C), jnp.float32),
    )(o2)


def _rsqrt16(d):
    i = lax.bitcast_convert_type(d, jnp.int32)
    i = 0x5F3759DF - lax.shift_right_arithmetic(i, 1)
    y = lax.bitcast_convert_type(i, jnp.float32)
    for _ in range(3):
        y = y * (1.5 - 0.5 * d * y * y)
    return y


def _prop_body(h_hbm, src_hbm, dst_hbm, out_hbm,
               t_sh, u_sh, deg_sh,
               src_v, dst_v, rowbuf, ones_v,
               nodebuf, outbuf, dqx, sdx, scal1, scal2, gsem, ssem):
    c = lax.axis_index("c")
    s = lax.axis_index("s")
    base = s * CHUNK

    pltpu.sync_copy(src_hbm.at[s], src_v)
    pltpu.sync_copy(dst_hbm.at[s], dst_v)

    def fill16(i, _):
        scal1[pl.ds(i * 16, 16)] = jnp.full((16,), 1.0, jnp.float32)
        return 0
    lax.fori_loop(0, CHUNK // 16, fill16, 0)

    def fillones(i, _):
        ones_v[pl.ds(i * 16, 16)] = jnp.full((16,), 1.0, jnp.float32)
        return 0
    lax.fori_loop(0, EBLK // 16, fillones, 0)

    pltpu.sync_copy(scal1, deg_sh.at[pl.ds(base, CHUNK)])
    plsc.subcore_barrier()

    def degblk(j, _):
        pltpu.sync_copy(ones_v, deg_sh.at[dst_v.at[j]], add=True)
        return 0
    lax.fori_loop(0, NBLK, degblk, 0)
    plsc.subcore_barrier()

    pltpu.sync_copy(deg_sh.at[pl.ds(base, CHUNK)], scal1)

    def newton(i, _):
        d = scal1[pl.ds(i * 16, 16)]
        y = _rsqrt16(d)
        scal1[pl.ds(i * 16, 16)] = y * y
        scal2[pl.ds(i * 16, 16)] = d * y
        return 0
    lax.fori_loop(0, CHUNK // 16, newton, 0)

    def expand(i, _):
        v1 = scal1[pl.ds(i * 16, 16)]
        v2 = scal2[pl.ds(i * 16, 16)]
        for l in range(16):
            dqx[i * 16 + l] = jnp.full((HALF,), v1[l], jnp.float32)
            sdx[i * 16 + l] = jnp.full((HALF,), v2[l], jnp.float32)
        return 0
    lax.fori_loop(0, CHUNK // 16, expand, 0)

    pltpu.sync_copy(h_hbm.at[c, pl.ds(base, CHUNK)], nodebuf)

    def init_row(r, _):
        hrow = nodebuf[r]
        dinv = dqx[r] * sdx[r]
        nodebuf[r] = hrow * dinv
        outbuf[r] = hrow * G[0]
        return 0
    lax.fori_loop(0, CHUNK, init_row, 0)

    pltpu.sync_copy(nodebuf, t_sh.at[pl.ds(base, CHUNK)])
    pltpu.sync_copy(nodebuf, u_sh.at[pl.ds(base, CHUNK)])
    plsc.subcore_barrier()

    for k in range(1, K + 1):
        for p in range(AHEAD):
            pltpu.async_copy(t_sh.at[src_v.at[p]], rowbuf.at[p], gsem)

        def edge(j, _):
            @pl.when(j >= AHEAD)
            def _():
                jj = j - AHEAD
                pltpu.make_async_copy(
                    rowbuf.at[jj % NBUF],
                    u_sh.at[dst_v.at[jj]], ssem).wait()

            @pl.when(j + AHEAD < NBLK)
            def _():
                jn = j + AHEAD
                pltpu.async_copy(
                    t_sh.at[src_v.at[jn]], rowbuf.at[jn % NBUF], gsem)

            pltpu.make_async_copy(
                t_sh.at[src_v.at[j]], rowbuf.at[j % NBUF], gsem).wait()
            pltpu.async_copy(
                rowbuf.at[j % NBUF], u_sh.at[dst_v.at[j]], ssem, add=True)
            return 0
        lax.fori_loop(0, NBLK, edge, 0)
        for p in range(AHEAD):
            jj = NBLK - AHEAD + p
            pltpu.make_async_copy(
                rowbuf.at[jj % NBUF], u_sh.at[dst_v.at[jj]], ssem).wait()
        plsc.subcore_barrier()

        pltpu.sync_copy(u_sh.at[pl.ds(base, CHUNK)], nodebuf)
        gk = G[k]

        def node(r, _):
            trow = nodebuf[r] * dqx[r]
            nodebuf[r] = trow
            outbuf[r] = outbuf[r] + trow * sdx[r] * gk
            return 0
        lax.fori_loop(0, CHUNK, node, 0)

        pltpu.sync_copy(nodebuf, t_sh.at[pl.ds(base, CHUNK)])
        if k < K:
            pltpu.sync_copy(nodebuf, u_sh.at[pl.ds(base, CHUNK)])
        plsc.subcore_barrier()

    pltpu.sync_copy(outbuf, out_hbm.at[c, pl.ds(base, CHUNK)])


def _prop(h2, srcb, dstb):
    mesh = plsc.VectorSubcoreMesh(
        core_axis_name="c", subcore_axis_name="s",
        num_cores=NCORES, num_subcores=NTILES)
    f = pl.kernel(
        _prop_body,
        out_type=jax.ShapeDtypeStruct((NCORES, N_PAD, HALF), jnp.float32),
        mesh=mesh,
        compiler_params=pltpu.CompilerParams(use_tc_tiling_on_sc=False),
        scratch_types=[
            pltpu.VMEM_SHARED((N_PAD, HALF), jnp.float32),
            pltpu.VMEM_SHARED((N_PAD, HALF), jnp.float32),
            pltpu.VMEM_SHARED((N_PAD,), jnp.float32),
            pltpu.VMEM((NBLK, EBLK), jnp.int32),
            pltpu.VMEM((NBLK, EBLK), jnp.int32),
            pltpu.VMEM((NBUF, EBLK, HALF), jnp.float32),
            pltpu.VMEM((EBLK,), jnp.float32),
            pltpu.VMEM((CHUNK, HALF), jnp.float32),
            pltpu.VMEM((CHUNK, HALF), jnp.float32),
            pltpu.VMEM((CHUNK, HALF), jnp.float32),
            pltpu.VMEM((CHUNK, HALF), jnp.float32),
            pltpu.VMEM((CHUNK,), jnp.float32),
            pltpu.VMEM((CHUNK,), jnp.float32),
            pltpu.SemaphoreType.DMA,
            pltpu.SemaphoreType.DMA,
        ],
    )
    return f(h2, srcb, dstb)


def kernel(x, edge_index, W1, b1, W2, b2):
    h2 = _mlp(x, W1, b1, W2, b2)

    pad_ids = N + (jnp.arange(E_PAD - E, dtype=jnp.int32) % (N_PAD - N))
    srcb = jnp.concatenate([edge_index[0], pad_ids]).reshape(NTILES, NBLK, EBLK)
    dstb = jnp.concatenate([edge_index[1], pad_ids]).reshape(NTILES, NBLK, EBLK)

    out2 = _prop(h2, srcb, dstb)
    return _lsm(out2)


# speedup vs baseline: 60.6543x; 1.0036x over previous
import functools
import math

import jax
import jax.numpy as jnp
from jax import lax
from jax.experimental import pallas as pl
from jax.experimental.pallas import tpu as pltpu
from jax.experimental.pallas import tpu_sc as plsc

N = 10000
D = 128
E = 320000
HIDDEN = 64
C = 32
K = 10
ALPHA = 1.0
A_ = 1.0
B_ = 0.5
CC = 0.5

NTILES = 16
NCORES = 2
HALF = C // NCORES
N_PAD = 10240
CHUNK = N_PAD // NTILES
EBLK = 128
NBLK = 157
E_PAD = NTILES * NBLK * EBLK
ROWBLK = 512
NBUF = 8
AHEAD = NBUF // 2


def _gammas():
    cs = [1.0, (A_ - B_) / A_]
    for n in range(1, K):
        cs.append(((n + A_ - B_) * cs[n] - n * cs[n - 1]) / A_)
    return [math.exp(-ALPHA) * (ALPHA ** k) / math.factorial(k) * cs[k] * CC
            for k in range(K + 1)]


G = _gammas()


def _mlp_body(x_ref, w1_ref, b1_ref, w2_ref, b2_ref, o_ref):
    i = pl.program_id(0)
    h = jnp.dot(x_ref[...], w1_ref[...], preferred_element_type=jnp.float32)
    h = jnp.maximum(h + b1_ref[...], 0.0)
    h = jnp.dot(h, w2_ref[...], preferred_element_type=jnp.float32)
    h = h + b2_ref[...]
    rows = i * ROWBLK + lax.broadcasted_iota(jnp.int32, (ROWBLK, C), 0)
    h = jnp.where(rows < N, h, 0.0)
    o_ref[0] = h[:, :HALF]
    o_ref[1] = h[:, HALF:]


def _mlp(x, W1, b1, W2, b2):
    grid = N_PAD // ROWBLK
    return pl.pallas_call(
        _mlp_body,
        grid=(grid,),
        in_specs=[
            pl.BlockSpec((ROWBLK, D), lambda i: (i, 0)),
            pl.BlockSpec((D, HIDDEN), lambda i: (0, 0)),
            pl.BlockSpec((1, HIDDEN), lambda i: (0, 0)),
            pl.BlockSpec((HIDDEN, C), lambda i: (0, 0)),
            pl.BlockSpec((1, C), lambda i: (0, 0)),
        ],
        out_specs=pl.BlockSpec((NCORES, ROWBLK, HALF), lambda i: (0, i, 0)),
        out_shape=jax.ShapeDtypeStruct((NCORES, N_PAD, HALF), jnp.float32),
    )(x, W1, b1.reshape(1, HIDDEN), W2, b2.reshape(1, C))


def _lsm_body(x_ref, o_ref):
    x = jnp.concatenate([x_ref[0], x_ref[1]], axis=1)
    m = jnp.max(x, axis=1, keepdims=True)
    s = jnp.sum(jnp.exp(x - m), axis=1, keepdims=True)
    o_ref[...] = x - m - jnp.log(s)


def _lsm(o2):
    grid = N_PAD // ROWBLK
    return pl.pallas_call(
        _lsm_body,
        grid=(grid,),
        in_specs=[pl.BlockSpec((NCORES, ROWBLK, HALF), lambda i: (0, i, 0))],
        out_specs=pl.BlockSpec((ROWBLK, C), lambda i: (i, 0)),
        out_shape=jax.ShapeDtypeStruct((N, C), jnp.float32),
    )(o2)


def _rsqrt16(d):
    i = lax.bitcast_convert_type(d, jnp.int32)
    i = 0x5F3759DF - lax.shift_right_arithmetic(i, 1)
    y = lax.bitcast_convert_type(i, jnp.float32)
    for _ in range(3):
        y = y * (1.5 - 0.5 * d * y * y)
    return y


def _prop_body(h_hbm, src_hbm, dst_hbm, out_hbm,
               t_sh, u_sh, deg_sh,
               src_v, dst_v, rowbuf, ones_v,
               nodebuf, outbuf, dqx, sdx, scal1, scal2, gsem, ssem):
    c = lax.axis_index("c")
    s = lax.axis_index("s")
    base = s * CHUNK

    pltpu.sync_copy(src_hbm.at[s], src_v)
    pltpu.sync_copy(dst_hbm.at[s], dst_v)

    def fill16(i, _):
        scal1[pl.ds(i * 16, 16)] = jnp.full((16,), 1.0, jnp.float32)
        return 0
    lax.fori_loop(0, CHUNK // 16, fill16, 0)

    def fillones(i, _):
        ones_v[pl.ds(i * 16, 16)] = jnp.full((16,), 1.0, jnp.float32)
        return 0
    lax.fori_loop(0, EBLK // 16, fillones, 0)

    pltpu.sync_copy(scal1, deg_sh.at[pl.ds(base, CHUNK)])
    plsc.subcore_barrier()

    def degblk(j, _):
        pltpu.sync_copy(ones_v, deg_sh.at[dst_v.at[j]], add=True)
        return 0
    lax.fori_loop(0, NBLK, degblk, 0)
    plsc.subcore_barrier()

    pltpu.sync_copy(deg_sh.at[pl.ds(base, CHUNK)], scal1)

    def newton(i, _):
        d = scal1[pl.ds(i * 16, 16)]
        y = _rsqrt16(d)
        scal1[pl.ds(i * 16, 16)] = y * y
        scal2[pl.ds(i * 16, 16)] = d * y
        return 0
    lax.fori_loop(0, CHUNK // 16, newton, 0)

    def expand(i, _):
        v1 = scal1[pl.ds(i * 16, 16)]
        v2 = scal2[pl.ds(i * 16, 16)]
        for l in range(16):
            dqx[i * 16 + l] = jnp.full((HALF,), v1[l], jnp.float32)
            sdx[i * 16 + l] = jnp.full((HALF,), v2[l], jnp.float32)
        return 0
    lax.fori_loop(0, CHUNK // 16, expand, 0)

    pltpu.sync_copy(h_hbm.at[c, pl.ds(base, CHUNK)], nodebuf)

    def init_row(r, _):
        hrow = nodebuf[r]
        dinv = dqx[r] * sdx[r]
        nodebuf[r] = hrow * dinv
        outbuf[r] = hrow * G[0]
        return 0
    lax.fori_loop(0, CHUNK, init_row, 0)

    pltpu.sync_copy(nodebuf, t_sh.at[pl.ds(base, CHUNK)])
    pltpu.sync_copy(nodebuf, u_sh.at[pl.ds(base, CHUNK)])
    plsc.subcore_barrier()

    for k in range(1, K + 1):
        for p in range(AHEAD):
            pltpu.async_copy(t_sh.at[src_v.at[p]], rowbuf.at[p], gsem)

        def edge(j, _):
            @pl.when(j >= AHEAD)
            def _():
                jj = j - AHEAD
                pltpu.make_async_copy(
                    rowbuf.at[jj % NBUF],
                    u_sh.at[dst_v.at[jj]], ssem).wait()

            @pl.when(j + AHEAD < NBLK)
            def _():
                jn = j + AHEAD
                pltpu.async_copy(
                    t_sh.at[src_v.at[jn]], rowbuf.at[jn % NBUF], gsem)

            pltpu.make_async_copy(
                t_sh.at[src_v.at[j]], rowbuf.at[j % NBUF], gsem).wait()
            pltpu.async_copy(
                rowbuf.at[j % NBUF], u_sh.at[dst_v.at[j]], ssem, add=True)
            return 0
        lax.fori_loop(0, NBLK, edge, 0)
        for p in range(AHEAD):
            jj = NBLK - AHEAD + p
            pltpu.make_async_copy(
                rowbuf.at[jj % NBUF], u_sh.at[dst_v.at[jj]], ssem).wait()
        plsc.subcore_barrier()

        pltpu.sync_copy(u_sh.at[pl.ds(base, CHUNK)], nodebuf)
        gk = G[k]

        def node(r, _):
            trow = nodebuf[r] * dqx[r]
            nodebuf[r] = trow
            outbuf[r] = outbuf[r] + trow * sdx[r] * gk
            return 0
        lax.fori_loop(0, CHUNK, node, 0)

        pltpu.sync_copy(nodebuf, t_sh.at[pl.ds(base, CHUNK)])
        if k < K:
            pltpu.sync_copy(nodebuf, u_sh.at[pl.ds(base, CHUNK)])
        plsc.subcore_barrier()

    pltpu.sync_copy(outbuf, out_hbm.at[c, pl.ds(base, CHUNK)])


def _prop(h2, srcb, dstb):
    mesh = plsc.VectorSubcoreMesh(
        core_axis_name="c", subcore_axis_name="s",
        num_cores=NCORES, num_subcores=NTILES)
    f = pl.kernel(
        _prop_body,
        out_type=jax.ShapeDtypeStruct((NCORES, N_PAD, HALF), jnp.float32),
        mesh=mesh,
        compiler_params=pltpu.CompilerParams(use_tc_tiling_on_sc=False),
        scratch_types=[
            pltpu.VMEM_SHARED((N_PAD, HALF), jnp.float32),
            pltpu.VMEM_SHARED((N_PAD, HALF), jnp.float32),
            pltpu.VMEM_SHARED((N_PAD,), jnp.float32),
            pltpu.VMEM((NBLK, EBLK), jnp.int32),
            pltpu.VMEM((NBLK, EBLK), jnp.int32),
            pltpu.VMEM((NBUF, EBLK, HALF), jnp.float32),
            pltpu.VMEM((EBLK,), jnp.float32),
            pltpu.VMEM((CHUNK, HALF), jnp.float32),
            pltpu.VMEM((CHUNK, HALF), jnp.float32),
            pltpu.VMEM((CHUNK, HALF), jnp.float32),
            pltpu.VMEM((CHUNK, HALF), jnp.float32),
            pltpu.VMEM((CHUNK,), jnp.float32),
            pltpu.VMEM((CHUNK,), jnp.float32),
            pltpu.SemaphoreType.DMA,
            pltpu.SemaphoreType.DMA,
        ],
    )
    return f(h2, srcb, dstb)


def kernel(x, edge_index, W1, b1, W2, b2):
    h2 = _mlp(x, W1, b1, W2, b2)

    pad_ids = N + (jnp.arange(E_PAD - E, dtype=jnp.int32) % (N_PAD - N))
    srcb = jnp.concatenate([edge_index[0], pad_ids]).reshape(NTILES, NBLK, EBLK)
    dstb = jnp.concatenate([edge_index[1], pad_ids]).reshape(NTILES, NBLK, EBLK)

    out2 = _prop(h2, srcb, dstb)
    return _lsm(out2)
